# baseline (speedup 1.0000x reference)
import jax
import jax.numpy as jnp
from jax.experimental import pallas as pl
from jax.experimental.pallas import tpu as pltpu

_N = 32768
_B = 16
_D = 16
_T = 2048
_NT = _N // _T
_OUT = 256


def _fused_kernel(seg_ref, pts_ref, W1_ref, b1_ref, W2_ref, b2_ref, W3_ref,
                  b3_ref, Wh_ref, bh_ref, y_ref, acc_ref, cnt_ref):
    i = pl.program_id(0)

    @pl.when(i == 0)
    def _():
        acc_ref[...] = jnp.zeros_like(acc_ref)
        cnt_ref[...] = jnp.zeros_like(cnt_ref)

    x = pts_ref[...]
    h = jnp.dot(x, W1_ref[...], preferred_element_type=jnp.float32) + b1_ref[...]
    h = jnp.maximum(h, 0.0)
    h = jnp.dot(h, W2_ref[...], preferred_element_type=jnp.float32) + b2_ref[...]
    h = jnp.maximum(h, 0.0)
    f = jnp.dot(h, W3_ref[...], preferred_element_type=jnp.float32) + b3_ref[...]

    seg = seg_ref[0]
    bidx = jax.lax.broadcasted_iota(jnp.int32, (_B, 1), 0)
    mt = (seg == bidx).astype(jnp.float32)
    cnt_ref[...] += jnp.sum(mt, axis=1, keepdims=True)

    frep = jnp.concatenate([f] * _D, axis=1)
    fbig = jnp.concatenate(
        [jnp.broadcast_to(f[:, k:k + 1], (_T, _D)) for k in range(_D)],
        axis=1)
    u = fbig * frep

    acc_ref[...] += jnp.dot(mt, u, preferred_element_type=jnp.float32)

    @pl.when(i == _NT - 1)
    def _():
        maxc = jnp.max(cnt_ref[...])
        cov = acc_ref[...] / maxc
        p = jnp.sign(cov) * jnp.sqrt(jnp.abs(cov) + 1e-12)
        y = jnp.dot(p, Wh_ref[...], preferred_element_type=jnp.float32) + bh_ref[...]
        nrm = jnp.sqrt(jnp.sum(y * y, axis=1, keepdims=True))
        y_ref[...] = y / (nrm + 1e-12)


def kernel(points, segment_ids, W1, b1, W2, b2, W3, b3, Wh, bh):
    seg = segment_ids.astype(jnp.int32).reshape(_NT, 1, _T)
    out = pl.pallas_call(
        _fused_kernel,
        grid=(_NT,),
        in_specs=[
            pl.BlockSpec((1, 1, _T), lambda i: (i, 0, 0)),
            pl.BlockSpec((_T, 4), lambda i: (i, 0)),
            pl.BlockSpec((4, 64), lambda i: (0, 0)),
            pl.BlockSpec((1, 64), lambda i: (0, 0)),
            pl.BlockSpec((64, 64), lambda i: (0, 0)),
            pl.BlockSpec((1, 64), lambda i: (0, 0)),
            pl.BlockSpec((64, _D), lambda i: (0, 0)),
            pl.BlockSpec((1, _D), lambda i: (0, 0)),
            pl.BlockSpec((_D * _D, _OUT), lambda i: (0, 0)),
            pl.BlockSpec((1, _OUT), lambda i: (0, 0)),
        ],
        out_specs=pl.BlockSpec((_B, _OUT), lambda i: (0, 0)),
        out_shape=jax.ShapeDtypeStruct((_B, _OUT), jnp.float32),
        scratch_shapes=[
            pltpu.VMEM((_B, _OUT), jnp.float32),
            pltpu.VMEM((_B, 1), jnp.float32),
        ],
    )(seg, points, W1, b1.reshape(1, -1), W2, b2.reshape(1, -1), W3,
      b3.reshape(1, -1), Wh, bh.reshape(1, -1))
    return out


# speedup vs baseline: 2.5103x; 2.5103x over previous
import jax
import jax.numpy as jnp
from jax.experimental import pallas as pl
from jax.experimental.pallas import tpu as pltpu

_N = 32768
_B = 16
_D = 16
_T = 2048
_NT = _N // _T
_OUT = 256


def _fused_kernel(bounds_ref, seg_ref, pts_ref, W1_ref, b1_ref, W2_ref,
                  b2_ref, W3_ref, b3_ref, Wh_ref, bh_ref, y_ref, acc_ref,
                  cnt_ref):
    i = pl.program_id(0)

    @pl.when(i == 0)
    def _():
        acc_ref[...] = jnp.zeros_like(acc_ref)
        cnt_ref[...] = jnp.zeros_like(cnt_ref)

    x = pts_ref[...]
    h = jnp.dot(x, W1_ref[...], preferred_element_type=jnp.float32) + b1_ref[...]
    h = jnp.maximum(h, 0.0)
    h = jnp.dot(h, W2_ref[...], preferred_element_type=jnp.float32) + b2_ref[...]
    h = jnp.maximum(h, 0.0)
    h = jnp.dot(h, W3_ref[...], preferred_element_type=jnp.float32) + b3_ref[...]
    f = h[:, :_D]

    seg = seg_ref[...]
    onehot = (seg == jax.lax.broadcasted_iota(jnp.int32, (1, _B), 1)
              ).astype(jnp.float32)
    cnt_ref[...] += jnp.sum(onehot, axis=0, keepdims=True)

    lo = bounds_ref[i, 0]
    hi = bounds_ref[i, 1]

    for b in range(_B):
        @pl.when(jnp.logical_and(b >= lo, b <= hi))
        def _(b=b):
            fm = f * onehot[:, b:b + 1]
            c = jax.lax.dot_general(
                fm, f, (((0,), (0,)), ((), ())),
                preferred_element_type=jnp.float32)
            cflat = jnp.concatenate(
                [c[k:k + 1, :] for k in range(_D)], axis=1)
            acc_ref[b:b + 1, :] += cflat

    @pl.when(i == _NT - 1)
    def _():
        maxc = jnp.max(cnt_ref[...])
        cov = acc_ref[...] / maxc
        p = jnp.sign(cov) * jnp.sqrt(jnp.abs(cov) + 1e-12)
        y = jnp.dot(p, Wh_ref[...], preferred_element_type=jnp.float32) + bh_ref[...]
        nrm = jnp.sqrt(jnp.sum(y * y, axis=1, keepdims=True))
        y_ref[...] = y / (nrm + 1e-12)


def kernel(points, segment_ids, W1, b1, W2, b2, W3, b3, Wh, bh):
    seg = segment_ids.astype(jnp.int32)
    bounds = jnp.stack([seg[::_T], seg[_T - 1::_T]], axis=1)
    W3p = jnp.pad(W3, ((0, 0), (0, 64 - _D)))
    b3p = jnp.pad(b3, (0, 64 - _D)).reshape(1, -1)
    out = pl.pallas_call(
        _fused_kernel,
        grid=(_NT,),
        in_specs=[
            pl.BlockSpec(memory_space=pltpu.SMEM),
            pl.BlockSpec((_T, 1), lambda i: (i, 0)),
            pl.BlockSpec((_T, 4), lambda i: (i, 0)),
            pl.BlockSpec((4, 64), lambda i: (0, 0)),
            pl.BlockSpec((1, 64), lambda i: (0, 0)),
            pl.BlockSpec((64, 64), lambda i: (0, 0)),
            pl.BlockSpec((1, 64), lambda i: (0, 0)),
            pl.BlockSpec((64, 64), lambda i: (0, 0)),
            pl.BlockSpec((1, 64), lambda i: (0, 0)),
            pl.BlockSpec((_D * _D, _OUT), lambda i: (0, 0)),
            pl.BlockSpec((1, _OUT), lambda i: (0, 0)),
        ],
        out_specs=pl.BlockSpec((_B, _OUT), lambda i: (0, 0)),
        out_shape=jax.ShapeDtypeStruct((_B, _OUT), jnp.float32),
        scratch_shapes=[
            pltpu.VMEM((_B, _OUT), jnp.float32),
            pltpu.VMEM((1, _B), jnp.float32),
        ],
    )(bounds, seg.reshape(_N, 1), points, W1, b1.reshape(1, -1), W2,
      b2.reshape(1, -1), W3p, b3p, Wh, bh.reshape(1, -1))
    return out
